# baseline (speedup 1.0000x reference)
import functools

import jax
import jax.numpy as jnp
from jax.experimental import pallas as pl


def _mm(a, b):
    return jnp.matmul(a, b, preferred_element_type=jnp.float32)


def _bmm(a, b):
    return jax.lax.dot_general(
        a, b, (((2,), (1,)), ((0,), (0,))), preferred_element_type=jnp.float32
    )


def _body(su_ref, sv_ref, sa_ref, va_ref,
          gc1w_ref, gc1b_ref, gc2w_ref, gc2b_ref,
          w1su_ref, w1sv_ref, f1b_ref,
          f2w_ref, f2b_ref, f3w_ref, f3b_ref, f4w_ref, f4b_ref,
          out_ref, *, bb, n, np_, nfeat, nhid, ncls):
    gc1w = gc1w_ref[...]
    gc2w = gc2w_ref[...]
    gc1b = gc1b_ref[...].reshape(1, 1, nhid)
    gc2b = gc2b_ref[...].reshape(1, 1, ncls)
    pad = np_ - n

    def branch(x_ref, adj_ref):
        x = x_ref[...]
        adj = adj_ref[...]
        s1 = _mm(x.reshape(n * bb, nfeat), gc1w).reshape(n, bb, nhid)
        s1 = jnp.concatenate(
            [s1, jnp.zeros((pad, bb, nhid), jnp.float32)], axis=0)
        s1 = jnp.swapaxes(s1, 0, 1)
        adjp = jnp.concatenate(
            [adj, jnp.zeros((pad, n, bb), jnp.float32)], axis=0)
        adjp = jnp.concatenate(
            [adjp, jnp.zeros((np_, pad, bb), jnp.float32)], axis=1)
        adjp = jnp.transpose(adjp, (2, 0, 1))
        h = jnp.maximum(_bmm(adjp, s1) + gc1b, 0.0)
        s2 = _mm(h.reshape(bb * np_, nhid), gc2w).reshape(bb, np_, ncls)
        g = _bmm(adjp, s2) + gc2b
        gt = jnp.transpose(g, (0, 2, 1))
        return gt.reshape(bb, ncls * np_)

    dsu = branch(su_ref, sa_ref)
    dsv = branch(sv_ref, va_ref)
    h1 = jnp.maximum(
        _mm(dsu, w1su_ref[...]) + _mm(dsv, w1sv_ref[...]) + f1b_ref[...], 0.0)
    h2 = jnp.maximum(_mm(h1, f2w_ref[...]) + f2b_ref[...], 0.0)
    h3 = jnp.maximum(_mm(h2, f3w_ref[...]) + f3b_ref[...], 0.0)
    out_ref[...] = _mm(h3, f4w_ref[...]) + f4b_ref[...]


def kernel(solute, solvent, solute_adj, solvent_adj,
           gc1_w, gc1_b, gc2_w, gc2_b,
           fc1_w, fc1_b, fc2_w, fc2_b, fc3_w, fc3_b, fc4_w, fc4_b):
    b, n, nfeat = solute.shape
    nhid = gc1_w.shape[1]
    ncls = gc2_w.shape[1]
    np_ = 64
    bb = 128
    grid = (b // bb,)

    su_t = jnp.transpose(solute, (1, 0, 2))
    sv_t = jnp.transpose(solvent, (1, 0, 2))
    sa_t = jnp.transpose(solute_adj, (1, 2, 0))
    va_t = jnp.transpose(solvent_adj, (1, 2, 0))

    nfc = fc1_w.shape[1]
    w3 = fc1_w.reshape(2, n, ncls, nfc)
    w3 = jnp.pad(w3, ((0, 0), (0, np_ - n), (0, 0), (0, 0)))
    w3 = jnp.transpose(w3, (0, 2, 1, 3))
    w1su = w3[0].reshape(ncls * np_, nfc)
    w1sv = w3[1].reshape(ncls * np_, nfc)

    def row(v):
        return v.reshape(1, -1)

    fixed = lambda i: (0, 0)

    in_specs = [
        pl.BlockSpec((n, bb, nfeat), lambda i: (0, i, 0)),
        pl.BlockSpec((n, bb, nfeat), lambda i: (0, i, 0)),
        pl.BlockSpec((n, n, bb), lambda i: (0, 0, i)),
        pl.BlockSpec((n, n, bb), lambda i: (0, 0, i)),
        pl.BlockSpec(gc1_w.shape, fixed),
        pl.BlockSpec((1, nhid), fixed),
        pl.BlockSpec(gc2_w.shape, fixed),
        pl.BlockSpec((1, ncls), fixed),
        pl.BlockSpec((ncls * np_, nfc), fixed),
        pl.BlockSpec((ncls * np_, nfc), fixed),
        pl.BlockSpec((1, fc1_b.shape[0]), fixed),
        pl.BlockSpec(fc2_w.shape, fixed),
        pl.BlockSpec((1, fc2_b.shape[0]), fixed),
        pl.BlockSpec(fc3_w.shape, fixed),
        pl.BlockSpec((1, fc3_b.shape[0]), fixed),
        pl.BlockSpec(fc4_w.shape, fixed),
        pl.BlockSpec((1, fc4_b.shape[0]), fixed),
    ]

    body = functools.partial(_body, bb=bb, n=n, np_=np_, nfeat=nfeat,
                             nhid=nhid, ncls=ncls)
    return pl.pallas_call(
        body,
        grid=grid,
        in_specs=in_specs,
        out_specs=pl.BlockSpec((bb, 1), lambda i: (i, 0)),
        out_shape=jax.ShapeDtypeStruct((b, 1), jnp.float32),
    )(su_t, sv_t, sa_t, va_t,
      gc1_w, row(gc1_b), gc2_w, row(gc2_b),
      w1su, w1sv, row(fc1_b),
      fc2_w, row(fc2_b), fc3_w, row(fc3_b), fc4_w, row(fc4_b))


# speedup vs baseline: 1.0833x; 1.0833x over previous
import functools

import jax
import jax.numpy as jnp
from jax.experimental import pallas as pl


def _mm(a, b):
    return jnp.matmul(a, b, preferred_element_type=jnp.float32)


def _bmm(a, b):
    return jax.lax.dot_general(
        a, b, (((2,), (1,)), ((0,), (0,))), preferred_element_type=jnp.float32
    )


def _body(su_ref, sv_ref, sa_ref, va_ref,
          gc1w_ref, gc1b_ref, gc2w_ref, gc2b_ref,
          w1su_ref, w1sv_ref, f1b_ref,
          f2w_ref, f2b_ref, f3w_ref, f3b_ref, f4w_ref, f4b_ref,
          out_ref, *, bb, n, np_, nfeat, nhid, ncls):
    gc1w = gc1w_ref[...]
    gc2w = gc2w_ref[...]
    gc1b = gc1b_ref[...].reshape(1, 1, nhid)
    gc2b = gc2b_ref[...].reshape(1, 1, ncls)
    pad = np_ - n

    def branch(x_ref, adj_ref):
        x = x_ref[...]
        adj = adj_ref[...]
        s1 = _mm(x.reshape(n * bb, nfeat), gc1w).reshape(n, bb, nhid)
        s1 = jnp.swapaxes(s1, 0, 1)
        adjp = jnp.transpose(adj, (2, 0, 1))
        adjp = jnp.concatenate(
            [adjp, jnp.zeros((bb, pad, n), jnp.float32)], axis=1)
        h = jnp.maximum(_bmm(adjp, s1) + gc1b, 0.0)
        s2 = _mm(h.reshape(bb * np_, nhid), gc2w).reshape(bb, np_, ncls)
        g = _bmm(adjp, s2[:, :n, :]) + gc2b
        gt = jnp.transpose(g, (0, 2, 1))
        return gt.reshape(bb, ncls * np_)

    dsu = branch(su_ref, sa_ref)
    dsv = branch(sv_ref, va_ref)
    h1 = jnp.maximum(
        _mm(dsu, w1su_ref[...]) + _mm(dsv, w1sv_ref[...]) + f1b_ref[...], 0.0)
    h2 = jnp.maximum(_mm(h1, f2w_ref[...]) + f2b_ref[...], 0.0)
    h3 = jnp.maximum(_mm(h2, f3w_ref[...]) + f3b_ref[...], 0.0)
    out_ref[...] = _mm(h3, f4w_ref[...]) + f4b_ref[...]


def kernel(solute, solvent, solute_adj, solvent_adj,
           gc1_w, gc1_b, gc2_w, gc2_b,
           fc1_w, fc1_b, fc2_w, fc2_b, fc3_w, fc3_b, fc4_w, fc4_b):
    b, n, nfeat = solute.shape
    nhid = gc1_w.shape[1]
    ncls = gc2_w.shape[1]
    np_ = 64
    bb = 128
    grid = (b // bb,)

    su_t = jnp.transpose(solute, (1, 0, 2))
    sv_t = jnp.transpose(solvent, (1, 0, 2))
    sa_t = jnp.transpose(solute_adj, (1, 2, 0))
    va_t = jnp.transpose(solvent_adj, (1, 2, 0))

    nfc = fc1_w.shape[1]
    w3 = fc1_w.reshape(2, n, ncls, nfc)
    w3 = jnp.pad(w3, ((0, 0), (0, np_ - n), (0, 0), (0, 0)))
    w3 = jnp.transpose(w3, (0, 2, 1, 3))
    w1su = w3[0].reshape(ncls * np_, nfc)
    w1sv = w3[1].reshape(ncls * np_, nfc)

    def row(v):
        return v.reshape(1, -1)

    fixed = lambda i: (0, 0)

    in_specs = [
        pl.BlockSpec((n, bb, nfeat), lambda i: (0, i, 0)),
        pl.BlockSpec((n, bb, nfeat), lambda i: (0, i, 0)),
        pl.BlockSpec((n, n, bb), lambda i: (0, 0, i)),
        pl.BlockSpec((n, n, bb), lambda i: (0, 0, i)),
        pl.BlockSpec(gc1_w.shape, fixed),
        pl.BlockSpec((1, nhid), fixed),
        pl.BlockSpec(gc2_w.shape, fixed),
        pl.BlockSpec((1, ncls), fixed),
        pl.BlockSpec((ncls * np_, nfc), fixed),
        pl.BlockSpec((ncls * np_, nfc), fixed),
        pl.BlockSpec((1, fc1_b.shape[0]), fixed),
        pl.BlockSpec(fc2_w.shape, fixed),
        pl.BlockSpec((1, fc2_b.shape[0]), fixed),
        pl.BlockSpec(fc3_w.shape, fixed),
        pl.BlockSpec((1, fc3_b.shape[0]), fixed),
        pl.BlockSpec(fc4_w.shape, fixed),
        pl.BlockSpec((1, fc4_b.shape[0]), fixed),
    ]

    body = functools.partial(_body, bb=bb, n=n, np_=np_, nfeat=nfeat,
                             nhid=nhid, ncls=ncls)
    return pl.pallas_call(
        body,
        grid=grid,
        in_specs=in_specs,
        out_specs=pl.BlockSpec((bb, 1), lambda i: (i, 0)),
        out_shape=jax.ShapeDtypeStruct((b, 1), jnp.float32),
    )(su_t, sv_t, sa_t, va_t,
      gc1_w, row(gc1_b), gc2_w, row(gc2_b),
      w1su, w1sv, row(fc1_b),
      fc2_w, row(fc2_b), fc3_w, row(fc3_b), fc4_w, row(fc4_b))


# speedup vs baseline: 1.0907x; 1.0068x over previous
import functools

import jax
import jax.numpy as jnp
from jax.experimental import pallas as pl


def _mm(a, b):
    return jnp.matmul(a, b, preferred_element_type=jnp.float32)


def _bmm(a, b):
    return jax.lax.dot_general(
        a, b, (((2,), (1,)), ((0,), (0,))), preferred_element_type=jnp.float32
    )


def _body(su_ref, sv_ref, sa_ref, va_ref,
          gc1w_ref, gc1b_ref, gc2w_ref, gc2b_ref,
          w1_ref, f1b_ref,
          f2w_ref, f2b_ref, f3w_ref, f3b_ref, f4w_ref, f4b_ref,
          out_ref, *, bb, n, np_, nfeat, nhid, ncls):
    gc1w = gc1w_ref[...]
    gc2w = gc2w_ref[...]
    gc1b = gc1b_ref[...].reshape(1, 1, nhid)
    gc2b = gc2b_ref[...].reshape(1, 1, ncls)
    pad = np_ - n

    def branch(x_ref, adj_ref):
        x = x_ref[...]
        adj = adj_ref[...]
        s1 = _mm(x.reshape(n * bb, nfeat), gc1w).reshape(n, bb, nhid)
        s1 = jnp.swapaxes(s1, 0, 1)
        adjp = jnp.transpose(adj, (2, 0, 1))
        adjp = jnp.concatenate(
            [adjp, jnp.zeros((bb, pad, n), jnp.float32)], axis=1)
        h = jnp.maximum(_bmm(adjp, s1) + gc1b, 0.0)
        s2 = _mm(h.reshape(bb * np_, nhid), gc2w).reshape(bb, np_, ncls)
        g = _bmm(adjp, s2[:, :n, :]) + gc2b
        gt = jnp.transpose(g, (0, 2, 1))
        return gt.reshape(bb, ncls * np_)

    dsu = branch(su_ref, sa_ref)
    dsv = branch(sv_ref, va_ref)
    d = jnp.concatenate([dsu, dsv], axis=1)
    h1 = jnp.maximum(_mm(d, w1_ref[...]) + f1b_ref[...], 0.0)
    h2 = jnp.maximum(_mm(h1, f2w_ref[...]) + f2b_ref[...], 0.0)
    h3 = jnp.maximum(_mm(h2, f3w_ref[...]) + f3b_ref[...], 0.0)
    out_ref[...] = _mm(h3, f4w_ref[...]) + f4b_ref[...]


def kernel(solute, solvent, solute_adj, solvent_adj,
           gc1_w, gc1_b, gc2_w, gc2_b,
           fc1_w, fc1_b, fc2_w, fc2_b, fc3_w, fc3_b, fc4_w, fc4_b):
    b, n, nfeat = solute.shape
    nhid = gc1_w.shape[1]
    ncls = gc2_w.shape[1]
    np_ = 64
    bb = 128
    grid = (b // bb,)

    su_t = jnp.transpose(solute, (1, 0, 2))
    sv_t = jnp.transpose(solvent, (1, 0, 2))
    sa_t = jnp.transpose(solute_adj, (1, 2, 0))
    va_t = jnp.transpose(solvent_adj, (1, 2, 0))

    nfc = fc1_w.shape[1]
    w3 = fc1_w.reshape(2, n, ncls, nfc)
    w3 = jnp.pad(w3, ((0, 0), (0, np_ - n), (0, 0), (0, 0)))
    w3 = jnp.transpose(w3, (0, 2, 1, 3))
    w1 = w3.reshape(2 * ncls * np_, nfc)

    def row(v):
        return v.reshape(1, -1)

    fixed = lambda i: (0, 0)

    in_specs = [
        pl.BlockSpec((n, bb, nfeat), lambda i: (0, i, 0)),
        pl.BlockSpec((n, bb, nfeat), lambda i: (0, i, 0)),
        pl.BlockSpec((n, n, bb), lambda i: (0, 0, i)),
        pl.BlockSpec((n, n, bb), lambda i: (0, 0, i)),
        pl.BlockSpec(gc1_w.shape, fixed),
        pl.BlockSpec((1, nhid), fixed),
        pl.BlockSpec(gc2_w.shape, fixed),
        pl.BlockSpec((1, ncls), fixed),
        pl.BlockSpec((2 * ncls * np_, nfc), fixed),
        pl.BlockSpec((1, fc1_b.shape[0]), fixed),
        pl.BlockSpec(fc2_w.shape, fixed),
        pl.BlockSpec((1, fc2_b.shape[0]), fixed),
        pl.BlockSpec(fc3_w.shape, fixed),
        pl.BlockSpec((1, fc3_b.shape[0]), fixed),
        pl.BlockSpec(fc4_w.shape, fixed),
        pl.BlockSpec((1, fc4_b.shape[0]), fixed),
    ]

    body = functools.partial(_body, bb=bb, n=n, np_=np_, nfeat=nfeat,
                             nhid=nhid, ncls=ncls)
    return pl.pallas_call(
        body,
        grid=grid,
        in_specs=in_specs,
        out_specs=pl.BlockSpec((bb, 1), lambda i: (i, 0)),
        out_shape=jax.ShapeDtypeStruct((b, 1), jnp.float32),
    )(su_t, sv_t, sa_t, va_t,
      gc1_w, row(gc1_b), gc2_w, row(gc2_b),
      w1, row(fc1_b),
      fc2_w, row(fc2_b), fc3_w, row(fc3_b), fc4_w, row(fc4_b))
